# baseline (speedup 1.0000x reference)
import functools

import jax
import jax.numpy as jnp
from jax import lax
from jax.experimental import pallas as pl
from jax.experimental.pallas import tpu as pltpu
from jax.experimental.pallas import tpu_sc as plsc

NC = 2
NS = 16
NW = NC * NS

CHUNK = 256
BLK = 1000


def _sc_mesh():
    return plsc.VectorSubcoreMesh(
        core_axis_name="c", subcore_axis_name="s",
        num_cores=NC, num_subcores=NS)


def _make_deg_kernel(npad, epw):
    @functools.partial(
        pl.kernel,
        out_type=jax.ShapeDtypeStruct((NW, npad), jnp.float32),
        mesh=_sc_mesh(),
        compiler_params=pltpu.CompilerParams(needs_layout_passes=False),
        scratch_types=[
            pltpu.VMEM((epw,), jnp.int32),
            pltpu.VMEM((npad,), jnp.float32),
        ],
    )
    def deg_kernel(dst_hbm, zeros_hbm, out_hbm, dstv, counts):
        cid = lax.axis_index("c")
        sid = lax.axis_index("s")
        wid = sid * NC + cid
        pltpu.sync_copy(dst_hbm.at[pl.ds(wid * epw, epw)], dstv)
        pltpu.sync_copy(zeros_hbm, counts)
        ones = jnp.ones((16,), jnp.float32)

        def body(i, carry):
            idx = dstv[pl.ds(i * 16, 16)]
            plsc.addupdate_scatter(counts, [idx], ones)
            return carry

        lax.fori_loop(0, epw // 16, body, 0)
        pltpu.sync_copy(counts, out_hbm.at[wid])

    return deg_kernel


def _make_scatter_kernel(n, npad, f, nchunks):
    nbuf = 4
    assert nchunks % nbuf == 0

    @functools.partial(
        pl.kernel,
        out_type=jax.ShapeDtypeStruct((NC, npad, f), jnp.float32),
        mesh=_sc_mesh(),
        compiler_params=pltpu.CompilerParams(
            needs_layout_passes=False, use_tc_tiling_on_sc=False),
        scratch_types=[
            pltpu.VMEM((nchunks, CHUNK), jnp.int32),
            pltpu.VMEM((nchunks, CHUNK), jnp.int32),
            pltpu.VMEM((nbuf, CHUNK, f), jnp.float32),
            pltpu.VMEM_SHARED((npad, f), jnp.float32),
            pltpu.SemaphoreType.DMA((nbuf,)),
            pltpu.SemaphoreType.DMA((nbuf,)),
        ],
    )
    def scatter_kernel(p_hbm, src2_hbm, dst2_hbm, zeros_hbm, out_hbm,
                       sidx, didx, rows, acc, gsem, ssem):
        cid = lax.axis_index("c")
        sid = lax.axis_index("s")
        wid = sid * NC + cid
        rpt = npad // NS
        pltpu.sync_copy(zeros_hbm.at[pl.ds(sid * rpt, rpt)],
                        acc.at[pl.ds(sid * rpt, rpt)])
        pltpu.sync_copy(src2_hbm.at[pl.ds(wid * nchunks, nchunks)], sidx)
        pltpu.sync_copy(dst2_hbm.at[pl.ds(wid * nchunks, nchunks)], didx)
        plsc.subcore_barrier()

        for b in range(nbuf):
            pltpu.async_copy(p_hbm.at[sidx.at[b]], rows.at[b], gsem.at[b])

        def body(g, carry):
            for b in range(nbuf):
                c = g * nbuf + b
                pltpu.make_async_copy(
                    p_hbm.at[sidx.at[c]], rows.at[b], gsem.at[b]).wait()
                pltpu.async_copy(
                    rows.at[b], acc.at[didx.at[c]], ssem.at[b], add=True)
                pb = (b - 1) % nbuf
                pc = c - 1

                @pl.when((pc >= 0) & (pc + nbuf < nchunks))
                def _():
                    pltpu.make_async_copy(
                        rows.at[pb], acc.at[didx.at[pc]], ssem.at[pb]).wait()
                    pltpu.async_copy(
                        p_hbm.at[sidx.at[pc + nbuf]], rows.at[pb],
                        gsem.at[pb])

            return carry

        lax.fori_loop(0, nchunks // nbuf, body, 0)
        for b in range(nbuf):
            pltpu.make_async_copy(
                rows.at[b], acc.at[didx.at[nchunks - nbuf + b]],
                ssem.at[b]).wait()
        plsc.subcore_barrier()
        pltpu.sync_copy(acc.at[pl.ds(sid * rpt, rpt)],
                        out_hbm.at[cid, pl.ds(sid * rpt, rpt)])

    return scatter_kernel


def _dinv_of(degt_blk):
    deg = jnp.sum(degt_blk, axis=1, keepdims=True) + 1.0
    return 1.0 / jnp.sqrt(deg)


def _tc1_body(x_ref, w1_ref, degt_ref, p1_ref):
    dinv = _dinv_of(degt_ref[...])
    xw = jnp.dot(x_ref[...], w1_ref[...], preferred_element_type=jnp.float32)
    p1_ref[...] = xw * dinv


def _tc2_body(a_ref, p1_ref, degt_ref, w2_ref, b1_ref, p2_ref):
    dinv = _dinv_of(degt_ref[...])
    acc = a_ref[0] + a_ref[1]
    h = jnp.maximum((acc + p1_ref[...]) * dinv + b1_ref[...], 0.0)
    p2_ref[...] = jnp.dot(h, w2_ref[...],
                          preferred_element_type=jnp.float32) * dinv


def _tc3_body(a_ref, p2_ref, degt_ref, b2_ref, y_ref):
    dinv = _dinv_of(degt_ref[...])
    y = (a_ref[0] + a_ref[1] + p2_ref[...]) * dinv + b2_ref[...]
    y_ref[...] = y[:, :y_ref.shape[1]]


def kernel(x, edge_index, W1, b1, W2, b2):
    n, in_ch = x.shape
    hid = W1.shape[1]
    out_ch = W2.shape[1]
    e = edge_index.shape[1]
    f2 = 8

    epw = ((e + NW * CHUNK - 1) // (NW * CHUNK)) * CHUNK
    epad = epw * NW
    nchunks = epw // CHUNK
    npad = ((n + 1 + NS * 8 - 1) // (NS * 8)) * (NS * 8)

    src = edge_index[0]
    dst = edge_index[1]
    pad = epad - e
    srcp = jnp.concatenate([src, jnp.zeros((pad,), jnp.int32)])
    dstp = jnp.concatenate([dst, jnp.full((pad,), n, jnp.int32)])
    src2 = srcp.reshape(epad // CHUNK, CHUNK)
    dst2 = dstp.reshape(epad // CHUNK, CHUNK)

    w2p = jnp.concatenate(
        [W2, jnp.zeros((hid, f2 - out_ch), jnp.float32)], axis=1)
    b1r = b1.reshape(1, hid)
    b2r = jnp.concatenate([b2, jnp.zeros((f2 - out_ch,), jnp.float32)])
    b2r = b2r.reshape(1, f2)

    zeros1 = jnp.zeros((npad,), jnp.float32)
    zeros64 = jnp.zeros((npad, hid), jnp.float32)
    zeros8 = jnp.zeros((npad, f2), jnp.float32)

    degp = _make_deg_kernel(npad, epw)(dstp, zeros1)
    degt = degp.T

    grid = n // BLK
    degt_spec = pl.BlockSpec((BLK, NW), lambda i: (i, 0))

    p1 = pl.pallas_call(
        _tc1_body,
        grid=(grid,),
        in_specs=[
            pl.BlockSpec((BLK, in_ch), lambda i: (i, 0)),
            pl.BlockSpec((in_ch, hid), lambda i: (0, 0)),
            degt_spec,
        ],
        out_specs=pl.BlockSpec((BLK, hid), lambda i: (i, 0)),
        out_shape=jax.ShapeDtypeStruct((n, hid), jnp.float32),
    )(x, W1, degt)

    acc1 = _make_scatter_kernel(n, npad, hid, nchunks)(p1, src2, dst2, zeros64)

    p2 = pl.pallas_call(
        _tc2_body,
        grid=(grid,),
        in_specs=[
            pl.BlockSpec((NC, BLK, hid), lambda i: (0, i, 0)),
            pl.BlockSpec((BLK, hid), lambda i: (i, 0)),
            degt_spec,
            pl.BlockSpec((hid, f2), lambda i: (0, 0)),
            pl.BlockSpec((1, hid), lambda i: (0, 0)),
        ],
        out_specs=pl.BlockSpec((BLK, f2), lambda i: (i, 0)),
        out_shape=jax.ShapeDtypeStruct((n, f2), jnp.float32),
    )(acc1, p1, degt, w2p, b1r)

    acc2 = _make_scatter_kernel(n, npad, f2, nchunks)(p2, src2, dst2, zeros8)

    out = pl.pallas_call(
        _tc3_body,
        grid=(grid,),
        in_specs=[
            pl.BlockSpec((NC, BLK, f2), lambda i: (0, i, 0)),
            pl.BlockSpec((BLK, f2), lambda i: (i, 0)),
            degt_spec,
            pl.BlockSpec((1, f2), lambda i: (0, 0)),
        ],
        out_specs=pl.BlockSpec((BLK, out_ch), lambda i: (i, 0)),
        out_shape=jax.ShapeDtypeStruct((n, out_ch), jnp.float32),
    )(acc2, p2, degt, b2r)

    return out


# speedup vs baseline: 1.3676x; 1.3676x over previous
import functools

import jax
import jax.numpy as jnp
from jax import lax
from jax.experimental import pallas as pl
from jax.experimental.pallas import tpu as pltpu
from jax.experimental.pallas import tpu_sc as plsc

NC = 2
NS = 16
NW = NC * NS

CHUNK = 256
BLK = 1000


def _sc_mesh():
    return plsc.VectorSubcoreMesh(
        core_axis_name="c", subcore_axis_name="s",
        num_cores=NC, num_subcores=NS)


def _make_deg_kernel(npad, epw):
    @functools.partial(
        pl.kernel,
        out_type=jax.ShapeDtypeStruct((NW, npad), jnp.float32),
        mesh=_sc_mesh(),
        compiler_params=pltpu.CompilerParams(needs_layout_passes=False),
        scratch_types=[
            pltpu.VMEM((epw,), jnp.int32),
            pltpu.VMEM((npad,), jnp.float32),
        ],
    )
    def deg_kernel(dst_hbm, zeros_hbm, out_hbm, dstv, counts):
        cid = lax.axis_index("c")
        sid = lax.axis_index("s")
        wid = sid * NC + cid
        pltpu.sync_copy(dst_hbm.at[pl.ds(wid * epw, epw)], dstv)
        pltpu.sync_copy(zeros_hbm, counts)
        ones = jnp.ones((16,), jnp.float32)

        def body(i, carry):
            idx = dstv[pl.ds(i * 16, 16)]
            plsc.addupdate_scatter(counts, [idx], ones)
            return carry

        lax.fori_loop(0, epw // 16, body, 0)
        pltpu.sync_copy(counts, out_hbm.at[wid])

    return deg_kernel


def _make_scatter_kernel(n, npad, f, nchunks):
    nbuf = 4
    assert nchunks % nbuf == 0

    @functools.partial(
        pl.kernel,
        out_type=jax.ShapeDtypeStruct((NC, npad, f), jnp.float32),
        mesh=_sc_mesh(),
        compiler_params=pltpu.CompilerParams(
            needs_layout_passes=False, use_tc_tiling_on_sc=False),
        scratch_types=[
            pltpu.VMEM((nchunks, CHUNK), jnp.int32),
            pltpu.VMEM((nchunks, CHUNK), jnp.int32),
            pltpu.VMEM((nbuf, CHUNK, f), jnp.float32),
            pltpu.VMEM_SHARED((npad, f), jnp.float32),
            pltpu.VMEM_SHARED((n, f), jnp.float32),
            pltpu.SemaphoreType.DMA((nbuf,)),
            pltpu.SemaphoreType.DMA((nbuf,)),
        ],
    )
    def scatter_kernel(p_hbm, src2_hbm, dst2_hbm, zeros_hbm, out_hbm,
                       sidx, didx, rows, acc, pspm, gsem, ssem):
        cid = lax.axis_index("c")
        sid = lax.axis_index("s")
        rpt = npad // NS
        spt = n // NS
        pltpu.sync_copy(zeros_hbm, acc.at[pl.ds(sid * rpt, rpt)])
        pltpu.sync_copy(p_hbm.at[cid, pl.ds(sid * spt, spt)],
                        pspm.at[pl.ds(sid * spt, spt)])
        pltpu.sync_copy(src2_hbm.at[pl.ds(sid * nchunks, nchunks)], sidx)
        pltpu.sync_copy(dst2_hbm.at[pl.ds(sid * nchunks, nchunks)], didx)
        plsc.subcore_barrier()

        for b in range(nbuf):
            pltpu.async_copy(pspm.at[sidx.at[b]], rows.at[b], gsem.at[b])

        def body(g, carry):
            for b in range(nbuf):
                c = g * nbuf + b
                pltpu.make_async_copy(
                    pspm.at[sidx.at[c]], rows.at[b], gsem.at[b]).wait()
                pltpu.async_copy(
                    rows.at[b], acc.at[didx.at[c]], ssem.at[b], add=True)
                pb = (b - 1) % nbuf
                pc = c - 1

                @pl.when((pc >= 0) & (pc + nbuf < nchunks))
                def _():
                    pltpu.make_async_copy(
                        rows.at[pb], acc.at[didx.at[pc]], ssem.at[pb]).wait()
                    pltpu.async_copy(
                        pspm.at[sidx.at[pc + nbuf]], rows.at[pb],
                        gsem.at[pb])

            return carry

        lax.fori_loop(0, nchunks // nbuf, body, 0)
        for b in range(nbuf):
            pltpu.make_async_copy(
                rows.at[b], acc.at[didx.at[nchunks - nbuf + b]],
                ssem.at[b]).wait()
        plsc.subcore_barrier()
        pltpu.sync_copy(acc.at[pl.ds(sid * rpt, rpt)],
                        out_hbm.at[cid, pl.ds(sid * rpt, rpt)])

    return scatter_kernel


def _dinv_of(degt_blk):
    deg = jnp.sum(degt_blk, axis=1, keepdims=True) + 1.0
    return 1.0 / jnp.sqrt(deg)


def _tc1_body(x_ref, w1_ref, degt_ref, p1_ref):
    dinv = _dinv_of(degt_ref[...])
    xw = jnp.dot(x_ref[...], w1_ref[...], preferred_element_type=jnp.float32)
    p1 = xw * dinv
    fh = p1_ref.shape[2]
    p1_ref[0] = p1[:, :fh]
    p1_ref[1] = p1[:, fh:]


def _tc2_body(a_ref, p1_ref, degt_ref, w2_ref, b1_ref, p2_ref):
    dinv = _dinv_of(degt_ref[...])
    acc = jnp.concatenate([a_ref[0], a_ref[1]], axis=1)
    p1 = jnp.concatenate([p1_ref[0], p1_ref[1]], axis=1)
    h = jnp.maximum((acc + p1) * dinv + b1_ref[...], 0.0)
    p2 = jnp.dot(h, w2_ref[...], preferred_element_type=jnp.float32) * dinv
    fh = p2_ref.shape[2]
    p2_ref[0] = p2[:, :fh]
    p2_ref[1] = p2[:, fh:]


def _tc3_body(a_ref, p2_ref, degt_ref, b2_ref, y_ref):
    dinv = _dinv_of(degt_ref[...])
    y = (a_ref[0] + p2_ref[0]) * dinv + b2_ref[...]
    y_ref[...] = y[:, :y_ref.shape[1]]


def kernel(x, edge_index, W1, b1, W2, b2):
    n, in_ch = x.shape
    hid = W1.shape[1]
    out_ch = W2.shape[1]
    e = edge_index.shape[1]
    f2 = 16
    fh1 = hid // NC
    fh2 = f2 // NC

    epw = ((e + NW * CHUNK - 1) // (NW * CHUNK)) * CHUNK
    epad = epw * NW
    nct = epad // CHUNK // NS
    npad = ((n + 1 + NS * 8 - 1) // (NS * 8)) * (NS * 8)

    src = edge_index[0]
    dst = edge_index[1]
    pad = epad - e
    srcp = jnp.concatenate([src, jnp.zeros((pad,), jnp.int32)])
    dstp = jnp.concatenate([dst, jnp.full((pad,), n, jnp.int32)])
    src2 = srcp.reshape(epad // CHUNK, CHUNK)
    dst2 = dstp.reshape(epad // CHUNK, CHUNK)

    w2p = jnp.concatenate(
        [W2, jnp.zeros((hid, f2 - out_ch), jnp.float32)], axis=1)
    b1r = b1.reshape(1, hid)
    b2r = jnp.concatenate([b2, jnp.zeros((fh2 - out_ch,), jnp.float32)])
    b2r = b2r.reshape(1, fh2)

    zeros1 = jnp.zeros((npad,), jnp.float32)
    zeros64 = jnp.zeros((npad // NS, fh1), jnp.float32)
    zeros8 = jnp.zeros((npad // NS, fh2), jnp.float32)

    degp = _make_deg_kernel(npad, epw)(dstp, zeros1)
    degt = degp.T

    grid = n // BLK
    degt_spec = pl.BlockSpec((BLK, NW), lambda i: (i, 0))

    p1 = pl.pallas_call(
        _tc1_body,
        grid=(grid,),
        in_specs=[
            pl.BlockSpec((BLK, in_ch), lambda i: (i, 0)),
            pl.BlockSpec((in_ch, hid), lambda i: (0, 0)),
            degt_spec,
        ],
        out_specs=pl.BlockSpec((NC, BLK, fh1), lambda i: (0, i, 0)),
        out_shape=jax.ShapeDtypeStruct((NC, n, fh1), jnp.float32),
    )(x, W1, degt)

    acc1 = _make_scatter_kernel(n, npad, fh1, nct)(p1, src2, dst2, zeros64)

    p2 = pl.pallas_call(
        _tc2_body,
        grid=(grid,),
        in_specs=[
            pl.BlockSpec((NC, BLK, fh1), lambda i: (0, i, 0)),
            pl.BlockSpec((NC, BLK, fh1), lambda i: (0, i, 0)),
            degt_spec,
            pl.BlockSpec((hid, f2), lambda i: (0, 0)),
            pl.BlockSpec((1, hid), lambda i: (0, 0)),
        ],
        out_specs=pl.BlockSpec((NC, BLK, fh2), lambda i: (0, i, 0)),
        out_shape=jax.ShapeDtypeStruct((NC, n, fh2), jnp.float32),
    )(acc1, p1, degt, w2p, b1r)

    acc2 = _make_scatter_kernel(n, npad, fh2, nct)(p2, src2, dst2, zeros8)

    out = pl.pallas_call(
        _tc3_body,
        grid=(grid,),
        in_specs=[
            pl.BlockSpec((1, BLK, fh2), lambda i: (0, i, 0)),
            pl.BlockSpec((1, BLK, fh2), lambda i: (0, i, 0)),
            degt_spec,
            pl.BlockSpec((1, fh2), lambda i: (0, 0)),
        ],
        out_specs=pl.BlockSpec((BLK, out_ch), lambda i: (i, 0)),
        out_shape=jax.ShapeDtypeStruct((n, out_ch), jnp.float32),
    )(acc2, p2, degt, b2r)

    return out


# speedup vs baseline: 1.4069x; 1.0288x over previous
import functools

import jax
import jax.numpy as jnp
from jax import lax
from jax.experimental import pallas as pl
from jax.experimental.pallas import tpu as pltpu
from jax.experimental.pallas import tpu_sc as plsc

NC = 2
NS = 16
NW = NC * NS

CHUNK = 256
BLK = 2000


def _sc_mesh():
    return plsc.VectorSubcoreMesh(
        core_axis_name="c", subcore_axis_name="s",
        num_cores=NC, num_subcores=NS)


def _make_deg_kernel(npad, epw):
    @functools.partial(
        pl.kernel,
        out_type=jax.ShapeDtypeStruct((NW, npad), jnp.float32),
        mesh=_sc_mesh(),
        compiler_params=pltpu.CompilerParams(needs_layout_passes=False),
        scratch_types=[
            pltpu.VMEM((epw,), jnp.int32),
            pltpu.VMEM((npad,), jnp.float32),
        ],
    )
    def deg_kernel(dst_hbm, zeros_hbm, out_hbm, dstv, counts):
        cid = lax.axis_index("c")
        sid = lax.axis_index("s")
        wid = sid * NC + cid
        pltpu.sync_copy(dst_hbm.at[pl.ds(wid * epw, epw)], dstv)
        pltpu.sync_copy(zeros_hbm, counts)
        ones = jnp.ones((16,), jnp.float32)

        def body(i, carry):
            idx = dstv[pl.ds(i * 16, 16)]
            plsc.addupdate_scatter(counts, [idx], ones)
            return carry

        lax.fori_loop(0, epw // 16, body, 0)
        pltpu.sync_copy(counts, out_hbm.at[wid])

    return deg_kernel


def _make_scatter_kernel(n, npad, f, nchunks):
    nbuf = 4
    assert nchunks % nbuf == 0

    @functools.partial(
        pl.kernel,
        out_type=jax.ShapeDtypeStruct((NC, npad, f), jnp.float32),
        mesh=_sc_mesh(),
        compiler_params=pltpu.CompilerParams(
            needs_layout_passes=False, use_tc_tiling_on_sc=False),
        scratch_types=[
            pltpu.VMEM((nchunks, CHUNK), jnp.int32),
            pltpu.VMEM((nchunks, CHUNK), jnp.int32),
            pltpu.VMEM((nbuf, CHUNK, f), jnp.float32),
            pltpu.VMEM_SHARED((npad, f), jnp.float32),
            pltpu.VMEM_SHARED((npad, f), jnp.float32),
            pltpu.SemaphoreType.DMA((nbuf,)),
            pltpu.SemaphoreType.DMA((nbuf,)),
        ],
    )
    def scatter_kernel(p_hbm, src2_hbm, dst2_hbm, zeros_hbm, out_hbm,
                       sidx, didx, rows, acc, pspm, gsem, ssem):
        cid = lax.axis_index("c")
        sid = lax.axis_index("s")
        rpt = npad // NS
        pltpu.sync_copy(zeros_hbm, acc.at[pl.ds(sid * rpt, rpt)])
        pltpu.sync_copy(p_hbm.at[cid, pl.ds(sid * rpt, rpt)],
                        pspm.at[pl.ds(sid * rpt, rpt)])
        pltpu.sync_copy(src2_hbm.at[pl.ds(sid * nchunks, nchunks)], sidx)
        pltpu.sync_copy(dst2_hbm.at[pl.ds(sid * nchunks, nchunks)], didx)
        plsc.subcore_barrier()

        for b in range(nbuf):
            pltpu.async_copy(pspm.at[sidx.at[b]], rows.at[b], gsem.at[b])

        def body(g, carry):
            for b in range(nbuf):
                c = g * nbuf + b
                pltpu.make_async_copy(
                    pspm.at[sidx.at[c]], rows.at[b], gsem.at[b]).wait()
                pltpu.async_copy(
                    rows.at[b], acc.at[didx.at[c]], ssem.at[b], add=True)
                pb = (b - 1) % nbuf
                pc = c - 1

                @pl.when((pc >= 0) & (pc + nbuf < nchunks))
                def _():
                    pltpu.make_async_copy(
                        rows.at[pb], acc.at[didx.at[pc]], ssem.at[pb]).wait()
                    pltpu.async_copy(
                        pspm.at[sidx.at[pc + nbuf]], rows.at[pb],
                        gsem.at[pb])

            return carry

        lax.fori_loop(0, nchunks // nbuf, body, 0)
        for b in range(nbuf):
            pltpu.make_async_copy(
                rows.at[b], acc.at[didx.at[nchunks - nbuf + b]],
                ssem.at[b]).wait()
        plsc.subcore_barrier()
        pltpu.sync_copy(acc.at[pl.ds(sid * rpt, rpt)],
                        out_hbm.at[cid, pl.ds(sid * rpt, rpt)])

    return scatter_kernel


def _dinv_of(degt_blk):
    deg = jnp.sum(degt_blk, axis=1, keepdims=True) + 1.0
    return 1.0 / jnp.sqrt(deg)


def _tc1_body(x_ref, w1_ref, degt_ref, p1_ref):
    dinv = _dinv_of(degt_ref[...])
    xw = jnp.dot(x_ref[...], w1_ref[...], preferred_element_type=jnp.float32)
    p1 = xw * dinv
    fh = p1_ref.shape[2]
    p1_ref[0] = p1[:, :fh]
    p1_ref[1] = p1[:, fh:]


def _tc2_body(a_ref, p1_ref, degt_ref, w2_ref, b1_ref, p2_ref):
    dinv = _dinv_of(degt_ref[...])
    acc = jnp.concatenate([a_ref[0], a_ref[1]], axis=1)
    p1 = jnp.concatenate([p1_ref[0], p1_ref[1]], axis=1)
    h = jnp.maximum((acc + p1) * dinv + b1_ref[...], 0.0)
    p2 = jnp.dot(h, w2_ref[...], preferred_element_type=jnp.float32) * dinv
    fh = p2_ref.shape[2]
    p2_ref[0] = p2[:, :fh]
    p2_ref[1] = p2[:, fh:]


def _tc3_body(a_ref, p2_ref, degt_ref, b2_ref, y_ref):
    dinv = _dinv_of(degt_ref[...])
    y = (a_ref[0] + p2_ref[0]) * dinv + b2_ref[...]
    y_ref[...] = y[:, :y_ref.shape[1]]


def kernel(x, edge_index, W1, b1, W2, b2):
    n, in_ch = x.shape
    hid = W1.shape[1]
    out_ch = W2.shape[1]
    e = edge_index.shape[1]
    f2 = 16
    fh1 = hid // NC
    fh2 = f2 // NC

    epw = ((e + NW * CHUNK - 1) // (NW * CHUNK)) * CHUNK
    epad = epw * NW
    nct = epad // CHUNK // NS
    npad = ((n + 1 + NS * 8 - 1) // (NS * 8)) * (NS * 8)

    src = edge_index[0]
    dst = edge_index[1]
    pad = epad - e
    srcp = jnp.concatenate([src, jnp.zeros((pad,), jnp.int32)])
    dstp = jnp.concatenate([dst, jnp.full((pad,), n, jnp.int32)])
    src2 = srcp.reshape(epad // CHUNK, CHUNK)
    dst2 = dstp.reshape(epad // CHUNK, CHUNK)

    w2p = jnp.concatenate(
        [W2, jnp.zeros((hid, f2 - out_ch), jnp.float32)], axis=1)
    b1r = b1.reshape(1, hid)
    b2r = jnp.concatenate([b2, jnp.zeros((fh2 - out_ch,), jnp.float32)])
    b2r = b2r.reshape(1, fh2)

    zeros1 = jnp.zeros((npad,), jnp.float32)
    zeros64 = jnp.zeros((npad // NS, fh1), jnp.float32)
    zeros8 = jnp.zeros((npad // NS, fh2), jnp.float32)

    degp = _make_deg_kernel(npad, epw)(dstp, zeros1)
    degt = degp.T

    grid = n // BLK
    degt_spec = pl.BlockSpec((BLK, NW), lambda i: (i, 0))

    p1 = pl.pallas_call(
        _tc1_body,
        grid=(grid,),
        in_specs=[
            pl.BlockSpec((BLK, in_ch), lambda i: (i, 0)),
            pl.BlockSpec((in_ch, hid), lambda i: (0, 0)),
            degt_spec,
        ],
        out_specs=pl.BlockSpec((NC, BLK, fh1), lambda i: (0, i, 0)),
        out_shape=jax.ShapeDtypeStruct((NC, npad, fh1), jnp.float32),
    )(x, W1, degt)

    acc1 = _make_scatter_kernel(n, npad, fh1, nct)(p1, src2, dst2, zeros64)

    p2 = pl.pallas_call(
        _tc2_body,
        grid=(grid,),
        in_specs=[
            pl.BlockSpec((NC, BLK, fh1), lambda i: (0, i, 0)),
            pl.BlockSpec((NC, BLK, fh1), lambda i: (0, i, 0)),
            degt_spec,
            pl.BlockSpec((hid, f2), lambda i: (0, 0)),
            pl.BlockSpec((1, hid), lambda i: (0, 0)),
        ],
        out_specs=pl.BlockSpec((NC, BLK, fh2), lambda i: (0, i, 0)),
        out_shape=jax.ShapeDtypeStruct((NC, npad, fh2), jnp.float32),
    )(acc1, p1, degt, w2p, b1r)

    acc2 = _make_scatter_kernel(n, npad, fh2, nct)(p2, src2, dst2, zeros8)

    out = pl.pallas_call(
        _tc3_body,
        grid=(grid,),
        in_specs=[
            pl.BlockSpec((1, BLK, fh2), lambda i: (0, i, 0)),
            pl.BlockSpec((1, BLK, fh2), lambda i: (0, i, 0)),
            degt_spec,
            pl.BlockSpec((1, fh2), lambda i: (0, 0)),
        ],
        out_specs=pl.BlockSpec((BLK, out_ch), lambda i: (i, 0)),
        out_shape=jax.ShapeDtypeStruct((n, out_ch), jnp.float32),
    )(acc2, p2, degt, b2r)

    return out


# speedup vs baseline: 1.5700x; 1.1159x over previous
import functools

import jax
import jax.numpy as jnp
from jax import lax
from jax.experimental import pallas as pl
from jax.experimental.pallas import tpu as pltpu
from jax.experimental.pallas import tpu_sc as plsc

NC = 2
NS = 16
NW = NC * NS

CHUNK = 128
BLK = 2048


def _sc_mesh():
    return plsc.VectorSubcoreMesh(
        core_axis_name="c", subcore_axis_name="s",
        num_cores=NC, num_subcores=NS)


def _make_deg_kernel(npad, epw):
    @functools.partial(
        pl.kernel,
        out_type=jax.ShapeDtypeStruct((NW, npad), jnp.float32),
        mesh=_sc_mesh(),
        compiler_params=pltpu.CompilerParams(needs_layout_passes=False),
        scratch_types=[
            pltpu.VMEM((epw,), jnp.int32),
            pltpu.VMEM((npad,), jnp.float32),
        ],
    )
    def deg_kernel(dst_hbm, zeros_hbm, out_hbm, dstv, counts):
        cid = lax.axis_index("c")
        sid = lax.axis_index("s")
        wid = sid * NC + cid
        pltpu.sync_copy(dst_hbm.at[pl.ds(wid * epw, epw)], dstv)
        pltpu.sync_copy(zeros_hbm, counts)
        ones = jnp.ones((16,), jnp.float32)

        def body(i, carry):
            idx = dstv[pl.ds(i * 16, 16)]
            plsc.addupdate_scatter(counts, [idx], ones)
            return carry

        lax.fori_loop(0, epw // 16, body, 0)
        pltpu.sync_copy(counts, out_hbm.at[wid])

    return deg_kernel


def _make_scatter_kernel(npad, f, nchunks):
    nbuf = 4
    assert nchunks % nbuf == 0

    @functools.partial(
        pl.kernel,
        out_type=jax.ShapeDtypeStruct((NC, npad, 128), jnp.float32),
        mesh=_sc_mesh(),
        compiler_params=pltpu.CompilerParams(
            needs_layout_passes=False, use_tc_tiling_on_sc=False),
        scratch_types=[
            pltpu.VMEM((nchunks, CHUNK), jnp.int32),
            pltpu.VMEM((nchunks, CHUNK), jnp.int32),
            pltpu.VMEM((nbuf, CHUNK, f), jnp.float32),
            pltpu.VMEM_SHARED((npad, f), jnp.float32),
            pltpu.VMEM_SHARED((npad, f), jnp.float32),
            pltpu.SemaphoreType.DMA((nbuf,)),
            pltpu.SemaphoreType.DMA((nbuf,)),
        ],
    )
    def scatter_kernel(p_hbm, src2_hbm, dst2_hbm, zeros_hbm, out_hbm,
                       sidx, didx, rows, acc, pspm, gsem, ssem):
        cid = lax.axis_index("c")
        sid = lax.axis_index("s")
        rpt = npad // NS
        pltpu.sync_copy(zeros_hbm.at[pl.ds(0, rpt), pl.ds(0, f)],
                        acc.at[pl.ds(sid * rpt, rpt)])
        pltpu.sync_copy(p_hbm.at[cid, pl.ds(sid * rpt, rpt), pl.ds(0, f)],
                        pspm.at[pl.ds(sid * rpt, rpt)])
        pltpu.sync_copy(src2_hbm.at[pl.ds(sid * nchunks, nchunks)], sidx)
        pltpu.sync_copy(dst2_hbm.at[pl.ds(sid * nchunks, nchunks)], didx)
        plsc.subcore_barrier()

        for b in range(nbuf):
            pltpu.async_copy(pspm.at[sidx.at[b]], rows.at[b], gsem.at[b])

        def body(g, carry):
            for b in range(nbuf):
                c = g * nbuf + b
                pltpu.make_async_copy(
                    pspm.at[sidx.at[c]], rows.at[b], gsem.at[b]).wait()
                pltpu.async_copy(
                    rows.at[b], acc.at[didx.at[c]], ssem.at[b], add=True)
                pb = (b - 1) % nbuf
                pc = c - 1

                @pl.when((pc >= 0) & (pc + nbuf < nchunks))
                def _():
                    pltpu.make_async_copy(
                        rows.at[pb], acc.at[didx.at[pc]], ssem.at[pb]).wait()
                    pltpu.async_copy(
                        pspm.at[sidx.at[pc + nbuf]], rows.at[pb],
                        gsem.at[pb])

            return carry

        lax.fori_loop(0, nchunks // nbuf, body, 0)
        for b in range(nbuf):
            pltpu.make_async_copy(
                rows.at[b], acc.at[didx.at[nchunks - nbuf + b]],
                ssem.at[b]).wait()
        plsc.subcore_barrier()
        pltpu.sync_copy(acc.at[pl.ds(sid * rpt, rpt)],
                        out_hbm.at[cid, pl.ds(sid * rpt, rpt), pl.ds(0, f)])

    return scatter_kernel


def _dinv_of(degt_blk):
    deg = jnp.sum(degt_blk, axis=1, keepdims=True) + 1.0
    return 1.0 / jnp.sqrt(deg)


def _tc1_body(x_ref, w1_ref, degt_ref, p1_ref):
    dinv = _dinv_of(degt_ref[...])
    xw = jnp.dot(x_ref[...], w1_ref[...], preferred_element_type=jnp.float32)
    p1 = xw * dinv
    fh = p1.shape[1] // NC
    z = jnp.zeros((p1.shape[0], 128 - fh), jnp.float32)
    p1_ref[0] = jnp.concatenate([p1[:, :fh], z], axis=1)
    p1_ref[1] = jnp.concatenate([p1[:, fh:], z], axis=1)


def _tc2_body(a_ref, p1_ref, degt_ref, w2_ref, b1_ref, p2_ref):
    dinv = _dinv_of(degt_ref[...])
    fh = w2_ref.shape[0] // NC
    acc = jnp.concatenate([a_ref[0][:, :fh], a_ref[1][:, :fh]], axis=1)
    p1 = jnp.concatenate([p1_ref[0][:, :fh], p1_ref[1][:, :fh]], axis=1)
    h = jnp.maximum((acc + p1) * dinv + b1_ref[...], 0.0)
    p2 = jnp.dot(h, w2_ref[...], preferred_element_type=jnp.float32) * dinv
    fh2 = p2.shape[1] // NC
    z = jnp.zeros((p2.shape[0], 128 - fh2), jnp.float32)
    p2_ref[0] = jnp.concatenate([p2[:, :fh2], z], axis=1)
    p2_ref[1] = jnp.concatenate([p2[:, fh2:], z], axis=1)


def _tc3_body(a_ref, p2_ref, degt_ref, b2_ref, y_ref):
    dinv = _dinv_of(degt_ref[...])
    fh2 = b2_ref.shape[1]
    y = (a_ref[0][:, :fh2] + p2_ref[0][:, :fh2]) * dinv + b2_ref[...]
    y_ref[...] = y[:, :y_ref.shape[1]]


def kernel(x, edge_index, W1, b1, W2, b2):
    n, in_ch = x.shape
    hid = W1.shape[1]
    out_ch = W2.shape[1]
    e = edge_index.shape[1]
    f2 = 16
    fh1 = hid // NC
    fh2 = f2 // NC

    epw = ((e + NW * CHUNK - 1) // (NW * CHUNK)) * CHUNK
    epad = epw * NW
    nct = epad // CHUNK // NS
    npad = ((n + 1 + NS * 16 - 1) // (NS * 16)) * (NS * 16)

    src = edge_index[0]
    dst = edge_index[1]
    pad = epad - e
    srcp = jnp.concatenate([src, jnp.zeros((pad,), jnp.int32)])
    dstp = jnp.concatenate([dst, jnp.full((pad,), n, jnp.int32)])
    src2 = srcp.reshape(epad // 128, 128)
    dst2 = dstp.reshape(epad // 128, 128)

    w2p = jnp.concatenate(
        [W2, jnp.zeros((hid, f2 - out_ch), jnp.float32)], axis=1)
    b1r = b1.reshape(1, hid)
    b2r = jnp.concatenate([b2, jnp.zeros((fh2 - out_ch,), jnp.float32)])
    b2r = b2r.reshape(1, fh2)

    zeros1 = jnp.zeros((npad,), jnp.float32)
    zerosb = jnp.zeros((npad // NS, 128), jnp.float32)

    degp = _make_deg_kernel(npad, epw)(dstp, zeros1)
    degt = degp.T

    grid = npad // BLK
    degt_spec = pl.BlockSpec((BLK, NW), lambda i: (i, 0))

    p1 = pl.pallas_call(
        _tc1_body,
        grid=(grid,),
        in_specs=[
            pl.BlockSpec((BLK, in_ch), lambda i: (i, 0)),
            pl.BlockSpec((in_ch, hid), lambda i: (0, 0)),
            degt_spec,
        ],
        out_specs=pl.BlockSpec((NC, BLK, 128), lambda i: (0, i, 0)),
        out_shape=jax.ShapeDtypeStruct((NC, npad, 128), jnp.float32),
    )(x, W1, degt)

    acc1 = _make_scatter_kernel(npad, fh1, nct)(p1, src2, dst2, zerosb)

    p2 = pl.pallas_call(
        _tc2_body,
        grid=(grid,),
        in_specs=[
            pl.BlockSpec((NC, BLK, 128), lambda i: (0, i, 0)),
            pl.BlockSpec((NC, BLK, 128), lambda i: (0, i, 0)),
            degt_spec,
            pl.BlockSpec((hid, f2), lambda i: (0, 0)),
            pl.BlockSpec((1, hid), lambda i: (0, 0)),
        ],
        out_specs=pl.BlockSpec((NC, BLK, 128), lambda i: (0, i, 0)),
        out_shape=jax.ShapeDtypeStruct((NC, npad, 128), jnp.float32),
    )(acc1, p1, degt, w2p, b1r)

    acc2 = _make_scatter_kernel(npad, fh2, nct)(p2, src2, dst2, zerosb)

    out = pl.pallas_call(
        _tc3_body,
        grid=(grid,),
        in_specs=[
            pl.BlockSpec((1, BLK, 128), lambda i: (0, i, 0)),
            pl.BlockSpec((1, BLK, 128), lambda i: (0, i, 0)),
            degt_spec,
            pl.BlockSpec((1, fh2), lambda i: (0, 0)),
        ],
        out_specs=pl.BlockSpec((BLK, out_ch), lambda i: (i, 0)),
        out_shape=jax.ShapeDtypeStruct((n, out_ch), jnp.float32),
    )(acc2, p2, degt, b2r)

    return out


# speedup vs baseline: 1.5730x; 1.0019x over previous
import functools

import jax
import jax.numpy as jnp
from jax import lax
from jax.experimental import pallas as pl
from jax.experimental.pallas import tpu as pltpu
from jax.experimental.pallas import tpu_sc as plsc

NC = 2
NS = 16
NW = NC * NS

CHUNK = 128
BLK = 2048


def _sc_mesh():
    return plsc.VectorSubcoreMesh(
        core_axis_name="c", subcore_axis_name="s",
        num_cores=NC, num_subcores=NS)


def _make_deg_kernel(npad, epw):
    @functools.partial(
        pl.kernel,
        out_type=jax.ShapeDtypeStruct((NW, npad), jnp.float32),
        mesh=_sc_mesh(),
        compiler_params=pltpu.CompilerParams(needs_layout_passes=False),
        scratch_types=[
            pltpu.VMEM((epw,), jnp.int32),
            pltpu.VMEM((npad,), jnp.float32),
        ],
    )
    def deg_kernel(dst_hbm, zeros_hbm, out_hbm, dstv, counts):
        cid = lax.axis_index("c")
        sid = lax.axis_index("s")
        wid = sid * NC + cid
        pltpu.sync_copy(dst_hbm.at[pl.ds(wid * epw, epw)], dstv)
        pltpu.sync_copy(zeros_hbm, counts)
        ones = jnp.ones((16,), jnp.float32)

        def body(i, carry):
            idx = dstv[pl.ds(i * 16, 16)]
            plsc.addupdate_scatter(counts, [idx], ones)
            return carry

        lax.fori_loop(0, epw // 16, body, 0)
        pltpu.sync_copy(counts, out_hbm.at[wid])

    return deg_kernel


def _make_scatter_kernel(npad, f, nchunks):
    nbuf = 8
    assert nchunks % nbuf == 0

    @functools.partial(
        pl.kernel,
        out_type=jax.ShapeDtypeStruct((NC, npad, 128), jnp.float32),
        mesh=_sc_mesh(),
        compiler_params=pltpu.CompilerParams(
            needs_layout_passes=False, use_tc_tiling_on_sc=False),
        scratch_types=[
            pltpu.VMEM((nchunks, CHUNK), jnp.int32),
            pltpu.VMEM((nchunks, CHUNK), jnp.int32),
            pltpu.VMEM((nbuf, CHUNK, f), jnp.float32),
            pltpu.VMEM_SHARED((npad, f), jnp.float32),
            pltpu.VMEM_SHARED((npad, f), jnp.float32),
            pltpu.SemaphoreType.DMA((nbuf,)),
            pltpu.SemaphoreType.DMA((nbuf,)),
        ],
    )
    def scatter_kernel(p_hbm, src2_hbm, dst2_hbm, zeros_hbm, out_hbm,
                       sidx, didx, rows, acc, pspm, gsem, ssem):
        cid = lax.axis_index("c")
        sid = lax.axis_index("s")
        rpt = npad // NS
        pltpu.sync_copy(zeros_hbm.at[pl.ds(0, rpt), pl.ds(0, f)],
                        acc.at[pl.ds(sid * rpt, rpt)])
        pltpu.sync_copy(p_hbm.at[cid, pl.ds(sid * rpt, rpt), pl.ds(0, f)],
                        pspm.at[pl.ds(sid * rpt, rpt)])
        pltpu.sync_copy(src2_hbm.at[pl.ds(sid * nchunks, nchunks)], sidx)
        pltpu.sync_copy(dst2_hbm.at[pl.ds(sid * nchunks, nchunks)], didx)
        plsc.subcore_barrier()

        for b in range(nbuf):
            pltpu.async_copy(pspm.at[sidx.at[b]], rows.at[b], gsem.at[b])

        def body(g, carry):
            for b in range(nbuf):
                c = g * nbuf + b
                pltpu.make_async_copy(
                    pspm.at[sidx.at[c]], rows.at[b], gsem.at[b]).wait()
                pltpu.async_copy(
                    rows.at[b], acc.at[didx.at[c]], ssem.at[b], add=True)
                pb = (b - 1) % nbuf
                pc = c - 1

                @pl.when((pc >= 0) & (pc + nbuf < nchunks))
                def _():
                    pltpu.make_async_copy(
                        rows.at[pb], acc.at[didx.at[pc]], ssem.at[pb]).wait()
                    pltpu.async_copy(
                        pspm.at[sidx.at[pc + nbuf]], rows.at[pb],
                        gsem.at[pb])

            return carry

        lax.fori_loop(0, nchunks // nbuf, body, 0)
        for b in range(nbuf):
            pltpu.make_async_copy(
                rows.at[b], acc.at[didx.at[nchunks - nbuf + b]],
                ssem.at[b]).wait()
        plsc.subcore_barrier()
        pltpu.sync_copy(acc.at[pl.ds(sid * rpt, rpt)],
                        out_hbm.at[cid, pl.ds(sid * rpt, rpt), pl.ds(0, f)])

    return scatter_kernel


def _dinv_of(degt_blk):
    deg = jnp.sum(degt_blk, axis=1, keepdims=True) + 1.0
    return 1.0 / jnp.sqrt(deg)


def _tc1_body(x_ref, w1_ref, degt_ref, p1_ref):
    dinv = _dinv_of(degt_ref[...])
    xw = jnp.dot(x_ref[...], w1_ref[...], preferred_element_type=jnp.float32)
    p1 = xw * dinv
    fh = p1.shape[1] // NC
    z = jnp.zeros((p1.shape[0], 128 - fh), jnp.float32)
    p1_ref[0] = jnp.concatenate([p1[:, :fh], z], axis=1)
    p1_ref[1] = jnp.concatenate([p1[:, fh:], z], axis=1)


def _tc2_body(a_ref, p1_ref, degt_ref, w2_ref, b1_ref, p2_ref):
    dinv = _dinv_of(degt_ref[...])
    fh = w2_ref.shape[0] // NC
    acc = jnp.concatenate([a_ref[0][:, :fh], a_ref[1][:, :fh]], axis=1)
    p1 = jnp.concatenate([p1_ref[0][:, :fh], p1_ref[1][:, :fh]], axis=1)
    h = jnp.maximum((acc + p1) * dinv + b1_ref[...], 0.0)
    p2 = jnp.dot(h, w2_ref[...], preferred_element_type=jnp.float32) * dinv
    fh2 = p2.shape[1] // NC
    z = jnp.zeros((p2.shape[0], 128 - fh2), jnp.float32)
    p2_ref[0] = jnp.concatenate([p2[:, :fh2], z], axis=1)
    p2_ref[1] = jnp.concatenate([p2[:, fh2:], z], axis=1)


def _tc3_body(a_ref, p2_ref, degt_ref, b2_ref, y_ref):
    dinv = _dinv_of(degt_ref[...])
    fh2 = b2_ref.shape[1]
    y = (a_ref[0][:, :fh2] + p2_ref[0][:, :fh2]) * dinv + b2_ref[...]
    y_ref[...] = y[:, :y_ref.shape[1]]


def kernel(x, edge_index, W1, b1, W2, b2):
    n, in_ch = x.shape
    hid = W1.shape[1]
    out_ch = W2.shape[1]
    e = edge_index.shape[1]
    f2 = 16
    fh1 = hid // NC
    fh2 = f2 // NC

    epw = ((e + NW * CHUNK - 1) // (NW * CHUNK)) * CHUNK
    epad = epw * NW
    nct = epad // CHUNK // NS
    npad = ((n + 1 + NS * 16 - 1) // (NS * 16)) * (NS * 16)

    src = edge_index[0]
    dst = edge_index[1]
    pad = epad - e
    srcp = jnp.concatenate([src, jnp.zeros((pad,), jnp.int32)])
    dstp = jnp.concatenate([dst, jnp.full((pad,), n, jnp.int32)])
    src2 = srcp.reshape(epad // 128, 128)
    dst2 = dstp.reshape(epad // 128, 128)

    w2p = jnp.concatenate(
        [W2, jnp.zeros((hid, f2 - out_ch), jnp.float32)], axis=1)
    b1r = b1.reshape(1, hid)
    b2r = jnp.concatenate([b2, jnp.zeros((fh2 - out_ch,), jnp.float32)])
    b2r = b2r.reshape(1, fh2)

    zeros1 = jnp.zeros((npad,), jnp.float32)
    zerosb = jnp.zeros((npad // NS, 128), jnp.float32)

    degp = _make_deg_kernel(npad, epw)(dstp, zeros1)
    degt = degp.T

    grid = npad // BLK
    degt_spec = pl.BlockSpec((BLK, NW), lambda i: (i, 0))

    p1 = pl.pallas_call(
        _tc1_body,
        grid=(grid,),
        in_specs=[
            pl.BlockSpec((BLK, in_ch), lambda i: (i, 0)),
            pl.BlockSpec((in_ch, hid), lambda i: (0, 0)),
            degt_spec,
        ],
        out_specs=pl.BlockSpec((NC, BLK, 128), lambda i: (0, i, 0)),
        out_shape=jax.ShapeDtypeStruct((NC, npad, 128), jnp.float32),
    )(x, W1, degt)

    acc1 = _make_scatter_kernel(npad, fh1, nct)(p1, src2, dst2, zerosb)

    p2 = pl.pallas_call(
        _tc2_body,
        grid=(grid,),
        in_specs=[
            pl.BlockSpec((NC, BLK, 128), lambda i: (0, i, 0)),
            pl.BlockSpec((NC, BLK, 128), lambda i: (0, i, 0)),
            degt_spec,
            pl.BlockSpec((hid, f2), lambda i: (0, 0)),
            pl.BlockSpec((1, hid), lambda i: (0, 0)),
        ],
        out_specs=pl.BlockSpec((NC, BLK, 128), lambda i: (0, i, 0)),
        out_shape=jax.ShapeDtypeStruct((NC, npad, 128), jnp.float32),
    )(acc1, p1, degt, w2p, b1r)

    acc2 = _make_scatter_kernel(npad, fh2, nct)(p2, src2, dst2, zerosb)

    out = pl.pallas_call(
        _tc3_body,
        grid=(grid,),
        in_specs=[
            pl.BlockSpec((1, BLK, 128), lambda i: (0, i, 0)),
            pl.BlockSpec((1, BLK, 128), lambda i: (0, i, 0)),
            degt_spec,
            pl.BlockSpec((1, fh2), lambda i: (0, 0)),
        ],
        out_specs=pl.BlockSpec((BLK, out_ch), lambda i: (i, 0)),
        out_shape=jax.ShapeDtypeStruct((n, out_ch), jnp.float32),
    )(acc2, p2, degt, b2r)

    return out


# speedup vs baseline: 1.6358x; 1.0399x over previous
import functools

import jax
import jax.numpy as jnp
from jax import lax
from jax.experimental import pallas as pl
from jax.experimental.pallas import tpu as pltpu
from jax.experimental.pallas import tpu_sc as plsc

NC = 2
NS = 16
NW = NC * NS

CHUNK = 128
BLK = 2048


def _sc_mesh():
    return plsc.VectorSubcoreMesh(
        core_axis_name="c", subcore_axis_name="s",
        num_cores=NC, num_subcores=NS)


def _make_deg_kernel(npad, epw):
    @functools.partial(
        pl.kernel,
        out_type=jax.ShapeDtypeStruct((NW, npad), jnp.float32),
        mesh=_sc_mesh(),
        compiler_params=pltpu.CompilerParams(needs_layout_passes=False),
        scratch_types=[
            pltpu.VMEM((epw,), jnp.int32),
            pltpu.VMEM((npad,), jnp.float32),
        ],
    )
    def deg_kernel(dst_hbm, zeros_hbm, out_hbm, dstv, counts):
        cid = lax.axis_index("c")
        sid = lax.axis_index("s")
        wid = sid * NC + cid
        pltpu.sync_copy(dst_hbm.at[pl.ds(wid * epw, epw)], dstv)
        pltpu.sync_copy(zeros_hbm, counts)
        ones = jnp.ones((16,), jnp.float32)

        def body(i, carry):
            idx = dstv[pl.ds(i * 16, 16)]
            plsc.addupdate_scatter(counts, [idx], ones)
            return carry

        lax.fori_loop(0, epw // 16, body, 0)
        pltpu.sync_copy(counts, out_hbm.at[wid])

    return deg_kernel


def _make_scatter_kernel(npad, f, nchunks):
    nbuf = 8
    assert nchunks % nbuf == 0

    @functools.partial(
        pl.kernel,
        out_type=jax.ShapeDtypeStruct((NC, npad, 128), jnp.float32),
        mesh=_sc_mesh(),
        compiler_params=pltpu.CompilerParams(
            needs_layout_passes=False, use_tc_tiling_on_sc=False),
        scratch_types=[
            pltpu.VMEM((nchunks, CHUNK), jnp.int32),
            pltpu.VMEM((nchunks, CHUNK), jnp.int32),
            pltpu.VMEM((nbuf, CHUNK, f), jnp.float32),
            pltpu.VMEM_SHARED((npad, f), jnp.float32),
            pltpu.VMEM_SHARED((npad, f), jnp.float32),
            pltpu.SemaphoreType.DMA((nbuf,)),
            pltpu.SemaphoreType.DMA((nbuf,)),
        ],
    )
    def scatter_kernel(p_hbm, src2_hbm, dst2_hbm, out_hbm,
                       sidx, didx, rows, acc, pspm, gsem, ssem):
        cid = lax.axis_index("c")
        sid = lax.axis_index("s")
        rpt = npad // NS
        pltpu.sync_copy(p_hbm.at[cid, pl.ds(sid * rpt, rpt), pl.ds(0, f)],
                        acc.at[pl.ds(sid * rpt, rpt)])
        pltpu.sync_copy(p_hbm.at[cid, pl.ds(sid * rpt, rpt), pl.ds(0, f)],
                        pspm.at[pl.ds(sid * rpt, rpt)])
        pltpu.sync_copy(src2_hbm.at[pl.ds(sid * nchunks, nchunks)], sidx)
        pltpu.sync_copy(dst2_hbm.at[pl.ds(sid * nchunks, nchunks)], didx)
        plsc.subcore_barrier()

        for b in range(nbuf):
            pltpu.async_copy(pspm.at[sidx.at[b]], rows.at[b], gsem.at[b])

        def body(g, carry):
            for b in range(nbuf):
                c = g * nbuf + b
                pltpu.make_async_copy(
                    pspm.at[sidx.at[c]], rows.at[b], gsem.at[b]).wait()
                pltpu.async_copy(
                    rows.at[b], acc.at[didx.at[c]], ssem.at[b], add=True)
                pb = (b - 1) % nbuf
                pc = c - 1

                @pl.when((pc >= 0) & (pc + nbuf < nchunks))
                def _():
                    pltpu.make_async_copy(
                        rows.at[pb], acc.at[didx.at[pc]], ssem.at[pb]).wait()
                    pltpu.async_copy(
                        pspm.at[sidx.at[pc + nbuf]], rows.at[pb],
                        gsem.at[pb])

            return carry

        lax.fori_loop(0, nchunks // nbuf, body, 0)
        for b in range(nbuf):
            pltpu.make_async_copy(
                rows.at[b], acc.at[didx.at[nchunks - nbuf + b]],
                ssem.at[b]).wait()
        plsc.subcore_barrier()
        pltpu.sync_copy(acc.at[pl.ds(sid * rpt, rpt)],
                        out_hbm.at[cid, pl.ds(sid * rpt, rpt), pl.ds(0, f)])

    return scatter_kernel


def _dinv_of(degt_blk):
    deg = jnp.sum(degt_blk, axis=1, keepdims=True) + 1.0
    return 1.0 / jnp.sqrt(deg)


def _tc1_body(x_ref, w1_ref, degt_ref, p1_ref):
    dinv = _dinv_of(degt_ref[...])
    xw = jnp.dot(x_ref[...], w1_ref[...], preferred_element_type=jnp.float32)
    p1 = xw * dinv
    fh = p1.shape[1] // NC
    z = jnp.zeros((p1.shape[0], 128 - fh), jnp.float32)
    p1_ref[0] = jnp.concatenate([p1[:, :fh], z], axis=1)
    p1_ref[1] = jnp.concatenate([p1[:, fh:], z], axis=1)


def _tc2_body(a_ref, degt_ref, w2_ref, b1_ref, p2_ref):
    dinv = _dinv_of(degt_ref[...])
    fh = w2_ref.shape[0] // NC
    acc = jnp.concatenate([a_ref[0][:, :fh], a_ref[1][:, :fh]], axis=1)
    h = jnp.maximum(acc * dinv + b1_ref[...], 0.0)
    p2 = jnp.dot(h, w2_ref[...], preferred_element_type=jnp.float32) * dinv
    fh2 = p2.shape[1] // NC
    z = jnp.zeros((p2.shape[0], 128 - fh2), jnp.float32)
    p2_ref[0] = jnp.concatenate([p2[:, :fh2], z], axis=1)
    p2_ref[1] = jnp.concatenate([p2[:, fh2:], z], axis=1)


def _tc3_body(a_ref, degt_ref, b2_ref, y_ref):
    dinv = _dinv_of(degt_ref[...])
    fh2 = b2_ref.shape[1]
    y = a_ref[0][:, :fh2] * dinv + b2_ref[...]
    y_ref[...] = y[:, :y_ref.shape[1]]


def kernel(x, edge_index, W1, b1, W2, b2):
    n, in_ch = x.shape
    hid = W1.shape[1]
    out_ch = W2.shape[1]
    e = edge_index.shape[1]
    f2 = 16
    fh1 = hid // NC
    fh2 = f2 // NC

    epw = ((e + NW * CHUNK - 1) // (NW * CHUNK)) * CHUNK
    epad = epw * NW
    nct = epad // CHUNK // NS
    npad = ((n + 1 + NS * 16 - 1) // (NS * 16)) * (NS * 16)

    src = edge_index[0]
    dst = edge_index[1]
    pad = epad - e
    srcp = jnp.concatenate([src, jnp.zeros((pad,), jnp.int32)])
    dstp = jnp.concatenate([dst, jnp.full((pad,), n, jnp.int32)])
    src2 = srcp.reshape(epad // 128, 128)
    dst2 = dstp.reshape(epad // 128, 128)

    w2p = jnp.concatenate(
        [W2, jnp.zeros((hid, f2 - out_ch), jnp.float32)], axis=1)
    b1r = b1.reshape(1, hid)
    b2r = jnp.concatenate([b2, jnp.zeros((fh2 - out_ch,), jnp.float32)])
    b2r = b2r.reshape(1, fh2)

    zeros1 = jnp.zeros((npad,), jnp.float32)

    degp = _make_deg_kernel(npad, epw)(dstp, zeros1)
    degt = degp.T

    grid = npad // BLK
    degt_spec = pl.BlockSpec((BLK, NW), lambda i: (i, 0))

    p1 = pl.pallas_call(
        _tc1_body,
        grid=(grid,),
        in_specs=[
            pl.BlockSpec((BLK, in_ch), lambda i: (i, 0)),
            pl.BlockSpec((in_ch, hid), lambda i: (0, 0)),
            degt_spec,
        ],
        out_specs=pl.BlockSpec((NC, BLK, 128), lambda i: (0, i, 0)),
        out_shape=jax.ShapeDtypeStruct((NC, npad, 128), jnp.float32),
    )(x, W1, degt)

    acc1 = _make_scatter_kernel(npad, fh1, nct)(p1, src2, dst2)

    p2 = pl.pallas_call(
        _tc2_body,
        grid=(grid,),
        in_specs=[
            pl.BlockSpec((NC, BLK, 128), lambda i: (0, i, 0)),
            degt_spec,
            pl.BlockSpec((hid, f2), lambda i: (0, 0)),
            pl.BlockSpec((1, hid), lambda i: (0, 0)),
        ],
        out_specs=pl.BlockSpec((NC, BLK, 128), lambda i: (0, i, 0)),
        out_shape=jax.ShapeDtypeStruct((NC, npad, 128), jnp.float32),
    )(acc1, degt, w2p, b1r)

    acc2 = _make_scatter_kernel(npad, fh2, nct)(p2, src2, dst2)

    out = pl.pallas_call(
        _tc3_body,
        grid=(grid,),
        in_specs=[
            pl.BlockSpec((1, BLK, 128), lambda i: (0, i, 0)),
            degt_spec,
            pl.BlockSpec((1, fh2), lambda i: (0, 0)),
        ],
        out_specs=pl.BlockSpec((BLK, out_ch), lambda i: (i, 0)),
        out_shape=jax.ShapeDtypeStruct((n, out_ch), jnp.float32),
    )(acc2, degt, b2r)

    return out
